# baseline (speedup 1.0000x reference)
import functools

import jax
import jax.numpy as jnp
from jax import lax
from jax.experimental import pallas as pl
from jax.experimental.pallas import tpu as pltpu
from jax.experimental.pallas import tpu_sc as plsc

_NUM_CORES = 2
_NUM_SUBCORES = 16
_NW = _NUM_CORES * _NUM_SUBCORES


_CB = 7


@functools.partial(jax.jit, static_argnames=("nb", "d"))
def _sc_detile(x4, *, nb, d):
    d2 = d // 8
    per_w = nb // _NW
    n_chunks = per_w // _CB
    chunk_elems = _CB * 128 * d
    mesh = plsc.VectorSubcoreMesh(core_axis_name="c", subcore_axis_name="s")

    @functools.partial(
        pl.kernel,
        mesh=mesh,
        compiler_params=pltpu.CompilerParams(
            use_tc_tiling_on_sc=False, needs_layout_passes=False),
        out_type=jax.ShapeDtypeStruct((nb * 128 * d,), jnp.float32),
        scratch_types=[
            pltpu.VMEM((chunk_elems,), jnp.float32),
            pltpu.VMEM((chunk_elems,), jnp.float32),
            pltpu.VMEM((chunk_elems,), jnp.float32),
            pltpu.VMEM((chunk_elems,), jnp.float32),
            pltpu.SemaphoreType.DMA,
            pltpu.SemaphoreType.DMA,
            pltpu.SemaphoreType.DMA,
        ],
    )
    def k(x4_hbm, out_hbm, in0, in1, out0, out1, sem_in, sem_o0, sem_o1):
        wid = lax.axis_index("s") * _NUM_CORES + lax.axis_index("c")
        cb_base = wid * per_w
        lane = lax.broadcasted_iota(jnp.int32, (16,), 0)
        pattern = (lane // 8) * (_CB * 1024) + (lane % 8) * 128

        def start_in(j, buf):
            cb0 = cb_base + j * _CB
            for a in range(d2):
                pltpu.async_copy(x4_hbm.at[pl.ds((a * nb + cb0) * 1024, _CB * 1024)],
                                 buf.at[pl.ds(a * _CB * 1024, _CB * 1024)], sem_in)

        def wait_in(buf):
            for a in range(d2):
                pltpu.make_async_copy(x4_hbm.at[pl.ds(0, _CB * 1024)],
                                      buf.at[pl.ds(a * _CB * 1024, _CB * 1024)],
                                      sem_in).wait()

        def regroup(buf, obuf):
            def lb_body(lb, c):
                l0 = lb * 8
                for ci in range(_CB):
                    vecs = []
                    for li in range(8):
                        idx = pattern + (ci * 1024 + l0 + li)
                        vecs.append(plsc.load_gather(buf, [idx]))
                    for li in range(8):
                        obuf[pl.ds((ci * 128 + l0 + li) * d, d)] = vecs[li]
                return c
            lax.fori_loop(0, 16, lb_body, 0)

        def start_out(j, obuf, sem):
            cb0 = cb_base + j * _CB
            pltpu.async_copy(obuf, out_hbm.at[pl.ds(cb0 * 128 * d, chunk_elems)], sem)

        def wait_out(obuf, sem):
            pltpu.make_async_copy(obuf, out_hbm.at[pl.ds(0, chunk_elems)], sem).wait()

        start_in(0, in0)

        def pair_body(k2, carry):
            j0 = 2 * k2
            j1 = j0 + 1

            @pl.when(j1 < n_chunks)
            def _():
                start_in(j1, in1)
            wait_in(in0)

            @pl.when(j0 >= 2)
            def _():
                wait_out(out0, sem_o0)
            regroup(in0, out0)
            start_out(j0, out0, sem_o0)

            @pl.when(j1 < n_chunks)
            def _():
                @pl.when(j1 + 1 < n_chunks)
                def _():
                    start_in(j1 + 1, in0)
                wait_in(in1)

                @pl.when(j1 >= 2)
                def _():
                    wait_out(out1, sem_o1)
                regroup(in1, out1)
                start_out(j1, out1, sem_o1)
            return carry

        lax.fori_loop(0, (n_chunks + 1) // 2, pair_body, 0)
        wait_out(out0, sem_o0)
        if n_chunks >= 2:
            wait_out(out1, sem_o1)

    return k(x4)


@functools.partial(jax.jit, static_argnames=("n", "d", "chunk"))
def _sc_gather(ids, fm_v_lin, fm_w, *, n, d, chunk):
    fm_v = fm_v_lin.reshape(fm_v_lin.shape[0] // d, d)
    n_chunks = (n // _NW) // chunk
    per_w = n // _NW
    mesh = plsc.VectorSubcoreMesh(core_axis_name="c", subcore_axis_name="s")

    @functools.partial(
        pl.kernel,
        mesh=mesh,
        compiler_params=pltpu.CompilerParams(use_tc_tiling_on_sc=False),
        out_type=[
            jax.ShapeDtypeStruct((n, d), jnp.float32),
            jax.ShapeDtypeStruct((n,), jnp.float32),
        ],
        scratch_types=[
            pltpu.VMEM((chunk,), jnp.int32),
            pltpu.VMEM((chunk, d), jnp.float32),
            pltpu.VMEM((chunk,), jnp.float32),
            pltpu.SemaphoreType.DMA,
            pltpu.SemaphoreType.DMA,
        ],
    )
    def k(ids_hbm, fmv_hbm, fmw_hbm, emb_out, w_out, idx_v, rows_v, w_v, sem_r, sem_w):
        wid = lax.axis_index("s") * _NUM_CORES + lax.axis_index("c")
        base = wid * per_w
        for j in range(n_chunks):
            off = base + j * chunk
            pltpu.sync_copy(ids_hbm.at[pl.ds(off, chunk)], idx_v)
            cp_r = pltpu.async_copy(fmv_hbm.at[idx_v], rows_v, sem_r)
            cp_w = pltpu.async_copy(fmw_hbm.at[idx_v], w_v, sem_w)
            cp_r.wait()
            cp_w.wait()
            pltpu.sync_copy(rows_v, emb_out.at[pl.ds(off, chunk)])
            pltpu.sync_copy(w_v, w_out.at[pl.ds(off, chunk)])

    return k(ids, fm_v, fm_w)


def _tc_body(n_layers, f, d, emb_ref, vals_ref, w_ref, fmb_ref, *param_refs):
    out_ref = param_refs[-1]
    param_refs = param_refs[:-1]
    fd = f * d
    emb = emb_ref[...]
    vals = vals_ref[...]

    rr = lax.broadcasted_iota(jnp.int32, (f, fd), 0)
    cc = lax.broadcasted_iota(jnp.int32, (f, fd), 1)
    s_mat = (cc // d == rr).astype(jnp.float32)
    vrep = jnp.dot(vals, s_mat, preferred_element_type=jnp.float32)
    x = emb * vrep

    c2 = lax.broadcasted_iota(jnp.int32, (fd, d), 0)
    d2 = lax.broadcasted_iota(jnp.int32, (fd, d), 1)
    t_mat = (c2 % d == d2).astype(jnp.float32)
    sum_vec = jnp.dot(x, t_mat, preferred_element_type=jnp.float32)
    y_v = 0.5 * (jnp.sum(sum_vec * sum_vec, axis=1, keepdims=True)
                 - jnp.sum(x * x, axis=1, keepdims=True))

    y_w = jnp.sum(w_ref[...] * vals, axis=1, keepdims=True)

    h = x
    for i in range(n_layers):
        w_l, b_l, inv_l, sh_l = param_refs[4 * i:4 * i + 4]
        h = jnp.dot(h.astype(jnp.bfloat16), w_l[...].astype(jnp.bfloat16),
                    preferred_element_type=jnp.float32) + b_l[...]
        h = jnp.maximum(h, 0.0)
        h = h * inv_l[...] + sh_l[...]
    w_out_ref, b_out_ref = param_refs[4 * n_layers:4 * n_layers + 2]
    y_d = jnp.dot(h.astype(jnp.bfloat16), w_out_ref[...].astype(jnp.bfloat16),
                  preferred_element_type=jnp.float32) + b_out_ref[0, 0]

    y = fmb_ref[0, 0] + y_w + y_v + y_d
    out_ref[...] = jax.nn.sigmoid(y)


def _tc_mlp(emb, vals, w, fmb, layer_params, w_out, b_out, *, blk):
    b_total, fd = emb.shape
    f = vals.shape[1]
    d = fd // f
    n_layers = len(layer_params)
    grid = (b_total // blk,)

    def row_spec(width):
        return pl.BlockSpec((blk, width), lambda i: (i, 0))

    def full_spec(shape):
        return pl.BlockSpec(shape, lambda i: (0,) * len(shape))

    in_specs = [
        row_spec(fd),
        row_spec(f),
        row_spec(f),
        full_spec((1, 1)),
    ]
    args = [emb, vals, w, fmb]
    for (w_l, b_l, inv_l, sh_l) in layer_params:
        in_specs += [full_spec(w_l.shape), full_spec(b_l.shape),
                     full_spec(inv_l.shape), full_spec(sh_l.shape)]
        args += [w_l, b_l, inv_l, sh_l]
    in_specs += [full_spec(w_out.shape), full_spec(b_out.shape)]
    args += [w_out, b_out]

    return pl.pallas_call(
        functools.partial(_tc_body, n_layers, f, d),
        grid=grid,
        in_specs=in_specs,
        out_specs=pl.BlockSpec((blk, 1), lambda i: (i, 0)),
        out_shape=jax.ShapeDtypeStruct((b_total, 1), jnp.float32),
    )(*args)


def kernel(feat_ids, feat_vals, FM_B, FM_W, FM_V, params):
    b, f = feat_ids.shape
    v, d = FM_V.shape
    n = b * f

    ids = feat_ids.reshape(-1).astype(jnp.int32)
    nb = ((v + 128 * _NW * _CB - 1) // (128 * _NW * _CB)) * _NW * _CB
    v_pad = nb * 128
    fvp = jnp.pad(FM_V, ((0, v_pad - v), (0, 0)))
    x1 = fvp.T.reshape(d // 8, 8, nb, 128).transpose(0, 2, 1, 3).reshape(-1)
    table_lin = _sc_detile(x1, nb=nb, d=d)
    emb_flat, w_flat = _sc_gather(ids, table_lin, FM_W, n=n, d=d, chunk=1664)
    emb = emb_flat.reshape(b, f * d)
    w = w_flat.reshape(b, f)

    n_layers = sum(1 for k in params if k.startswith("W") and k != "W_out")
    layer_params = []
    for i in range(n_layers):
        inv = params[f"gamma{i}"] / jnp.sqrt(params[f"var{i}"] + 1e-3)
        sh = params[f"beta{i}"] - params[f"mean{i}"] * inv
        layer_params.append((
            params[f"W{i}"],
            params[f"b{i}"].reshape(1, -1),
            inv.reshape(1, -1),
            sh.reshape(1, -1),
        ))
    w_out = params["W_out"]
    b_out = params["b_out"].reshape(1, 1)
    fmb = FM_B.reshape(1, 1)

    pred = _tc_mlp(emb, feat_vals, w, fmb, layer_params, w_out, b_out, blk=512)
    return pred.reshape(-1)


# speedup vs baseline: 1.1259x; 1.1259x over previous
import functools

import jax
import jax.numpy as jnp
from jax import lax
from jax.experimental import pallas as pl
from jax.experimental.pallas import tpu as pltpu
from jax.experimental.pallas import tpu_sc as plsc

_NUM_CORES = 2
_NUM_SUBCORES = 16
_NW = _NUM_CORES * _NUM_SUBCORES


_CB = 7


@functools.partial(jax.jit, static_argnames=("nb", "d"))
def _sc_detile(x4, *, nb, d):
    d2 = d // 8
    per_w = nb // _NW
    n_chunks = per_w // _CB
    chunk_elems = _CB * 128 * d
    mesh = plsc.VectorSubcoreMesh(core_axis_name="c", subcore_axis_name="s")

    @functools.partial(
        pl.kernel,
        mesh=mesh,
        compiler_params=pltpu.CompilerParams(
            use_tc_tiling_on_sc=False, needs_layout_passes=False),
        out_type=jax.ShapeDtypeStruct((nb * 128 * d,), jnp.float32),
        scratch_types=[
            pltpu.VMEM((chunk_elems,), jnp.float32),
            pltpu.VMEM((chunk_elems,), jnp.float32),
            pltpu.VMEM((chunk_elems,), jnp.float32),
            pltpu.VMEM((chunk_elems,), jnp.float32),
            pltpu.SemaphoreType.DMA,
            pltpu.SemaphoreType.DMA,
            pltpu.SemaphoreType.DMA,
        ],
    )
    def k(x4_hbm, out_hbm, in0, in1, out0, out1, sem_in, sem_o0, sem_o1):
        wid = lax.axis_index("s") * _NUM_CORES + lax.axis_index("c")
        cb_base = wid * per_w
        lane16 = lax.broadcasted_iota(jnp.int32, (16,), 0) * d

        def start_in(j, buf):
            cb0 = cb_base + j * _CB
            for a in range(d2):
                pltpu.async_copy(x4_hbm.at[pl.ds((a * nb + cb0) * 1024, _CB * 1024)],
                                 buf.at[pl.ds(a * _CB * 1024, _CB * 1024)], sem_in)

        def wait_in(buf):
            for a in range(d2):
                pltpu.make_async_copy(x4_hbm.at[pl.ds(0, _CB * 1024)],
                                      buf.at[pl.ds(a * _CB * 1024, _CB * 1024)],
                                      sem_in).wait()

        def regroup(buf, obuf):
            for a in range(d2):
                for ci in range(_CB):
                    for s in range(8):
                        off = ((a * _CB + ci) * 8 + s) * 128
                        base = ci * (128 * d) + (8 * a + s)
                        vecs = [buf[pl.ds(off + lg * 16, 16)] for lg in range(8)]
                        for lg in range(8):
                            plsc.store_scatter(
                                obuf, [lane16 + (base + lg * 16 * d)], vecs[lg])

        def start_out(j, obuf, sem):
            cb0 = cb_base + j * _CB
            pltpu.async_copy(obuf, out_hbm.at[pl.ds(cb0 * 128 * d, chunk_elems)], sem)

        def wait_out(obuf, sem):
            pltpu.make_async_copy(obuf, out_hbm.at[pl.ds(0, chunk_elems)], sem).wait()

        start_in(0, in0)

        def pair_body(k2, carry):
            j0 = 2 * k2
            j1 = j0 + 1

            @pl.when(j1 < n_chunks)
            def _():
                start_in(j1, in1)
            wait_in(in0)

            @pl.when(j0 >= 2)
            def _():
                wait_out(out0, sem_o0)
            regroup(in0, out0)
            start_out(j0, out0, sem_o0)

            @pl.when(j1 < n_chunks)
            def _():
                @pl.when(j1 + 1 < n_chunks)
                def _():
                    start_in(j1 + 1, in0)
                wait_in(in1)

                @pl.when(j1 >= 2)
                def _():
                    wait_out(out1, sem_o1)
                regroup(in1, out1)
                start_out(j1, out1, sem_o1)
            return carry

        lax.fori_loop(0, (n_chunks + 1) // 2, pair_body, 0)
        wait_out(out0, sem_o0)
        if n_chunks >= 2:
            wait_out(out1, sem_o1)

    return k(x4)


@functools.partial(jax.jit, static_argnames=("n", "d", "chunk"))
def _sc_gather(ids, fm_v_lin, fm_w, *, n, d, chunk):
    fm_v = fm_v_lin.reshape(fm_v_lin.shape[0] // d, d)
    n_chunks = (n // _NW) // chunk
    per_w = n // _NW
    mesh = plsc.VectorSubcoreMesh(core_axis_name="c", subcore_axis_name="s")

    @functools.partial(
        pl.kernel,
        mesh=mesh,
        compiler_params=pltpu.CompilerParams(use_tc_tiling_on_sc=False),
        out_type=[
            jax.ShapeDtypeStruct((n, d), jnp.float32),
            jax.ShapeDtypeStruct((n,), jnp.float32),
        ],
        scratch_types=[
            pltpu.VMEM((chunk,), jnp.int32),
            pltpu.VMEM((chunk, d), jnp.float32),
            pltpu.VMEM((chunk,), jnp.float32),
            pltpu.SemaphoreType.DMA,
            pltpu.SemaphoreType.DMA,
        ],
    )
    def k(ids_hbm, fmv_hbm, fmw_hbm, emb_out, w_out, idx_v, rows_v, w_v, sem_r, sem_w):
        wid = lax.axis_index("s") * _NUM_CORES + lax.axis_index("c")
        base = wid * per_w
        for j in range(n_chunks):
            off = base + j * chunk
            pltpu.sync_copy(ids_hbm.at[pl.ds(off, chunk)], idx_v)
            cp_r = pltpu.async_copy(fmv_hbm.at[idx_v], rows_v, sem_r)
            cp_w = pltpu.async_copy(fmw_hbm.at[idx_v], w_v, sem_w)
            cp_r.wait()
            cp_w.wait()
            pltpu.sync_copy(rows_v, emb_out.at[pl.ds(off, chunk)])
            pltpu.sync_copy(w_v, w_out.at[pl.ds(off, chunk)])

    return k(ids, fm_v, fm_w)


def _tc_body(n_layers, f, d, emb_ref, vals_ref, w_ref, fmb_ref, *param_refs):
    out_ref = param_refs[-1]
    param_refs = param_refs[:-1]
    fd = f * d
    emb = emb_ref[...]
    vals = vals_ref[...]

    rr = lax.broadcasted_iota(jnp.int32, (f, fd), 0)
    cc = lax.broadcasted_iota(jnp.int32, (f, fd), 1)
    s_mat = (cc // d == rr).astype(jnp.float32)
    vrep = jnp.dot(vals, s_mat, preferred_element_type=jnp.float32)
    x = emb * vrep

    c2 = lax.broadcasted_iota(jnp.int32, (fd, d), 0)
    d2 = lax.broadcasted_iota(jnp.int32, (fd, d), 1)
    t_mat = (c2 % d == d2).astype(jnp.float32)
    sum_vec = jnp.dot(x, t_mat, preferred_element_type=jnp.float32)
    y_v = 0.5 * (jnp.sum(sum_vec * sum_vec, axis=1, keepdims=True)
                 - jnp.sum(x * x, axis=1, keepdims=True))

    y_w = jnp.sum(w_ref[...] * vals, axis=1, keepdims=True)

    h = x
    for i in range(n_layers):
        w_l, b_l, inv_l, sh_l = param_refs[4 * i:4 * i + 4]
        h = jnp.dot(h.astype(jnp.bfloat16), w_l[...].astype(jnp.bfloat16),
                    preferred_element_type=jnp.float32) + b_l[...]
        h = jnp.maximum(h, 0.0)
        h = h * inv_l[...] + sh_l[...]
    w_out_ref, b_out_ref = param_refs[4 * n_layers:4 * n_layers + 2]
    y_d = jnp.dot(h.astype(jnp.bfloat16), w_out_ref[...].astype(jnp.bfloat16),
                  preferred_element_type=jnp.float32) + b_out_ref[0, 0]

    y = fmb_ref[0, 0] + y_w + y_v + y_d
    out_ref[...] = jax.nn.sigmoid(y)


def _tc_mlp(emb, vals, w, fmb, layer_params, w_out, b_out, *, blk):
    b_total, fd = emb.shape
    f = vals.shape[1]
    d = fd // f
    n_layers = len(layer_params)
    grid = (b_total // blk,)

    def row_spec(width):
        return pl.BlockSpec((blk, width), lambda i: (i, 0))

    def full_spec(shape):
        return pl.BlockSpec(shape, lambda i: (0,) * len(shape))

    in_specs = [
        row_spec(fd),
        row_spec(f),
        row_spec(f),
        full_spec((1, 1)),
    ]
    args = [emb, vals, w, fmb]
    for (w_l, b_l, inv_l, sh_l) in layer_params:
        in_specs += [full_spec(w_l.shape), full_spec(b_l.shape),
                     full_spec(inv_l.shape), full_spec(sh_l.shape)]
        args += [w_l, b_l, inv_l, sh_l]
    in_specs += [full_spec(w_out.shape), full_spec(b_out.shape)]
    args += [w_out, b_out]

    return pl.pallas_call(
        functools.partial(_tc_body, n_layers, f, d),
        grid=grid,
        in_specs=in_specs,
        out_specs=pl.BlockSpec((blk, 1), lambda i: (i, 0)),
        out_shape=jax.ShapeDtypeStruct((b_total, 1), jnp.float32),
    )(*args)


def kernel(feat_ids, feat_vals, FM_B, FM_W, FM_V, params):
    b, f = feat_ids.shape
    v, d = FM_V.shape
    n = b * f

    ids = feat_ids.reshape(-1).astype(jnp.int32)
    nb = ((v + 128 * _NW * _CB - 1) // (128 * _NW * _CB)) * _NW * _CB
    v_pad = nb * 128
    fvp = jnp.pad(FM_V, ((0, v_pad - v), (0, 0)))
    x1 = fvp.T.reshape(d // 8, 8, nb, 128).transpose(0, 2, 1, 3).reshape(-1)
    table_lin = _sc_detile(x1, nb=nb, d=d)
    emb_flat, w_flat = _sc_gather(ids, table_lin, FM_W, n=n, d=d, chunk=1664)
    emb = emb_flat.reshape(b, f * d)
    w = w_flat.reshape(b, f)

    n_layers = sum(1 for k in params if k.startswith("W") and k != "W_out")
    layer_params = []
    for i in range(n_layers):
        inv = params[f"gamma{i}"] / jnp.sqrt(params[f"var{i}"] + 1e-3)
        sh = params[f"beta{i}"] - params[f"mean{i}"] * inv
        layer_params.append((
            params[f"W{i}"],
            params[f"b{i}"].reshape(1, -1),
            inv.reshape(1, -1),
            sh.reshape(1, -1),
        ))
    w_out = params["W_out"]
    b_out = params["b_out"].reshape(1, 1)
    fmb = FM_B.reshape(1, 1)

    pred = _tc_mlp(emb, feat_vals, w, fmb, layer_params, w_out, b_out, blk=512)
    return pred.reshape(-1)


# speedup vs baseline: 1.6315x; 1.4490x over previous
import functools

import jax
import jax.numpy as jnp
from jax import lax
from jax.experimental import pallas as pl
from jax.experimental.pallas import tpu as pltpu
from jax.experimental.pallas import tpu_sc as plsc

_NUM_CORES = 2
_NUM_SUBCORES = 16
_NW = _NUM_CORES * _NUM_SUBCORES


_CB = 7


@functools.partial(jax.jit, static_argnames=("nb", "d"))
def _sc_detile(x4, *, nb, d):
    d2 = d // 8
    per_w = nb // _NW
    n_chunks = per_w // _CB
    chunk_elems = _CB * 128 * d
    mesh = plsc.VectorSubcoreMesh(core_axis_name="c", subcore_axis_name="s")

    @functools.partial(
        pl.kernel,
        mesh=mesh,
        compiler_params=pltpu.CompilerParams(
            use_tc_tiling_on_sc=False, needs_layout_passes=False),
        out_type=jax.ShapeDtypeStruct((nb * 128 * d,), jnp.float32),
        scratch_types=[
            pltpu.VMEM((chunk_elems,), jnp.float32),
            pltpu.VMEM((chunk_elems,), jnp.float32),
            pltpu.VMEM((chunk_elems,), jnp.float32),
            pltpu.VMEM((chunk_elems,), jnp.float32),
            pltpu.SemaphoreType.DMA,
            pltpu.SemaphoreType.DMA,
            pltpu.SemaphoreType.DMA,
        ],
    )
    def k(x4_hbm, out_hbm, in0, in1, out0, out1, sem_in, sem_o0, sem_o1):
        wid = lax.axis_index("s") * _NUM_CORES + lax.axis_index("c")
        cb_base = wid * per_w
        lane16 = lax.broadcasted_iota(jnp.int32, (16,), 0) * d

        def start_in(j, buf):
            cb0 = cb_base + j * _CB
            for a in range(d2):
                pltpu.async_copy(x4_hbm.at[pl.ds((a * nb + cb0) * 1024, _CB * 1024)],
                                 buf.at[pl.ds(a * _CB * 1024, _CB * 1024)], sem_in)

        def wait_in(buf):
            for a in range(d2):
                pltpu.make_async_copy(x4_hbm.at[pl.ds(0, _CB * 1024)],
                                      buf.at[pl.ds(a * _CB * 1024, _CB * 1024)],
                                      sem_in).wait()

        def regroup(buf, obuf):
            @functools.partial(plsc.parallel_loop, 0, d2 * _CB * 8, unroll=2)
            def _(t):
                a = t // (_CB * 8)
                ci = (t // 8) % _CB
                s = t % 8
                off = t * 128
                base = ci * (128 * d) + (8 * a + s)
                vecs = [buf[pl.ds(off + lg * 16, 16)] for lg in range(8)]
                for lg in range(8):
                    plsc.store_scatter(
                        obuf, [lane16 + (base + lg * 16 * d)], vecs[lg])

        def start_out(j, obuf, sem):
            cb0 = cb_base + j * _CB
            pltpu.async_copy(obuf, out_hbm.at[pl.ds(cb0 * 128 * d, chunk_elems)], sem)

        def wait_out(obuf, sem):
            pltpu.make_async_copy(obuf, out_hbm.at[pl.ds(0, chunk_elems)], sem).wait()

        start_in(0, in0)

        def pair_body(k2, carry):
            j0 = 2 * k2
            j1 = j0 + 1

            @pl.when(j1 < n_chunks)
            def _():
                start_in(j1, in1)
            wait_in(in0)

            @pl.when(j0 >= 2)
            def _():
                wait_out(out0, sem_o0)
            regroup(in0, out0)
            start_out(j0, out0, sem_o0)

            @pl.when(j1 < n_chunks)
            def _():
                @pl.when(j1 + 1 < n_chunks)
                def _():
                    start_in(j1 + 1, in0)
                wait_in(in1)

                @pl.when(j1 >= 2)
                def _():
                    wait_out(out1, sem_o1)
                regroup(in1, out1)
                start_out(j1, out1, sem_o1)
            return carry

        lax.fori_loop(0, (n_chunks + 1) // 2, pair_body, 0)
        wait_out(out0, sem_o0)
        if n_chunks >= 2:
            wait_out(out1, sem_o1)

    return k(x4)


@functools.partial(jax.jit, static_argnames=("n", "d", "chunk"))
def _sc_gather(ids, fm_v_lin, fm_w, *, n, d, chunk):
    fm_v = fm_v_lin.reshape(fm_v_lin.shape[0] // d, d)
    n_chunks = (n // _NW) // chunk
    per_w = n // _NW
    mesh = plsc.VectorSubcoreMesh(core_axis_name="c", subcore_axis_name="s")

    @functools.partial(
        pl.kernel,
        mesh=mesh,
        compiler_params=pltpu.CompilerParams(use_tc_tiling_on_sc=False),
        out_type=[
            jax.ShapeDtypeStruct((n, d), jnp.float32),
            jax.ShapeDtypeStruct((n,), jnp.float32),
        ],
        scratch_types=[
            pltpu.VMEM((chunk,), jnp.int32),
            pltpu.VMEM((chunk, d), jnp.float32),
            pltpu.VMEM((chunk,), jnp.float32),
            pltpu.SemaphoreType.DMA,
            pltpu.SemaphoreType.DMA,
        ],
    )
    def k(ids_hbm, fmv_hbm, fmw_hbm, emb_out, w_out, idx_v, rows_v, w_v, sem_r, sem_w):
        wid = lax.axis_index("s") * _NUM_CORES + lax.axis_index("c")
        base = wid * per_w
        for j in range(n_chunks):
            off = base + j * chunk
            pltpu.sync_copy(ids_hbm.at[pl.ds(off, chunk)], idx_v)
            cp_r = pltpu.async_copy(fmv_hbm.at[idx_v], rows_v, sem_r)
            cp_w = pltpu.async_copy(fmw_hbm.at[idx_v], w_v, sem_w)
            cp_r.wait()
            cp_w.wait()
            pltpu.sync_copy(rows_v, emb_out.at[pl.ds(off, chunk)])
            pltpu.sync_copy(w_v, w_out.at[pl.ds(off, chunk)])

    return k(ids, fm_v, fm_w)


def _tc_body(n_layers, f, d, emb_ref, vals_ref, w_ref, fmb_ref, *param_refs):
    out_ref = param_refs[-1]
    param_refs = param_refs[:-1]
    fd = f * d
    emb = emb_ref[...]
    vals = vals_ref[...]

    rr = lax.broadcasted_iota(jnp.int32, (f, fd), 0)
    cc = lax.broadcasted_iota(jnp.int32, (f, fd), 1)
    s_mat = (cc // d == rr).astype(jnp.float32)
    vrep = jnp.dot(vals, s_mat, preferred_element_type=jnp.float32)
    x = emb * vrep

    c2 = lax.broadcasted_iota(jnp.int32, (fd, d), 0)
    d2 = lax.broadcasted_iota(jnp.int32, (fd, d), 1)
    t_mat = (c2 % d == d2).astype(jnp.float32)
    sum_vec = jnp.dot(x, t_mat, preferred_element_type=jnp.float32)
    y_v = 0.5 * (jnp.sum(sum_vec * sum_vec, axis=1, keepdims=True)
                 - jnp.sum(x * x, axis=1, keepdims=True))

    y_w = jnp.sum(w_ref[...] * vals, axis=1, keepdims=True)

    h = x
    for i in range(n_layers):
        w_l, b_l, inv_l, sh_l = param_refs[4 * i:4 * i + 4]
        h = jnp.dot(h.astype(jnp.bfloat16), w_l[...].astype(jnp.bfloat16),
                    preferred_element_type=jnp.float32) + b_l[...]
        h = jnp.maximum(h, 0.0)
        h = h * inv_l[...] + sh_l[...]
    w_out_ref, b_out_ref = param_refs[4 * n_layers:4 * n_layers + 2]
    y_d = jnp.dot(h.astype(jnp.bfloat16), w_out_ref[...].astype(jnp.bfloat16),
                  preferred_element_type=jnp.float32) + b_out_ref[0, 0]

    y = fmb_ref[0, 0] + y_w + y_v + y_d
    out_ref[...] = jax.nn.sigmoid(y)


def _tc_mlp(emb, vals, w, fmb, layer_params, w_out, b_out, *, blk):
    b_total, fd = emb.shape
    f = vals.shape[1]
    d = fd // f
    n_layers = len(layer_params)
    grid = (b_total // blk,)

    def row_spec(width):
        return pl.BlockSpec((blk, width), lambda i: (i, 0))

    def full_spec(shape):
        return pl.BlockSpec(shape, lambda i: (0,) * len(shape))

    in_specs = [
        row_spec(fd),
        row_spec(f),
        row_spec(f),
        full_spec((1, 1)),
    ]
    args = [emb, vals, w, fmb]
    for (w_l, b_l, inv_l, sh_l) in layer_params:
        in_specs += [full_spec(w_l.shape), full_spec(b_l.shape),
                     full_spec(inv_l.shape), full_spec(sh_l.shape)]
        args += [w_l, b_l, inv_l, sh_l]
    in_specs += [full_spec(w_out.shape), full_spec(b_out.shape)]
    args += [w_out, b_out]

    return pl.pallas_call(
        functools.partial(_tc_body, n_layers, f, d),
        grid=grid,
        in_specs=in_specs,
        out_specs=pl.BlockSpec((blk, 1), lambda i: (i, 0)),
        out_shape=jax.ShapeDtypeStruct((b_total, 1), jnp.float32),
    )(*args)


def kernel(feat_ids, feat_vals, FM_B, FM_W, FM_V, params):
    b, f = feat_ids.shape
    v, d = FM_V.shape
    n = b * f

    ids = feat_ids.reshape(-1).astype(jnp.int32)
    nb = ((v + 128 * _NW * _CB - 1) // (128 * _NW * _CB)) * _NW * _CB
    v_pad = nb * 128
    fvp = jnp.pad(FM_V, ((0, v_pad - v), (0, 0)))
    x1 = fvp.T.reshape(d // 8, 8, nb, 128).transpose(0, 2, 1, 3).reshape(-1)
    table_lin = _sc_detile(x1, nb=nb, d=d)
    emb_flat, w_flat = _sc_gather(ids, table_lin, FM_W, n=n, d=d, chunk=1664)
    emb = emb_flat.reshape(b, f * d)
    w = w_flat.reshape(b, f)

    n_layers = sum(1 for k in params if k.startswith("W") and k != "W_out")
    layer_params = []
    for i in range(n_layers):
        inv = params[f"gamma{i}"] / jnp.sqrt(params[f"var{i}"] + 1e-3)
        sh = params[f"beta{i}"] - params[f"mean{i}"] * inv
        layer_params.append((
            params[f"W{i}"],
            params[f"b{i}"].reshape(1, -1),
            inv.reshape(1, -1),
            sh.reshape(1, -1),
        ))
    w_out = params["W_out"]
    b_out = params["b_out"].reshape(1, 1)
    fmb = FM_B.reshape(1, 1)

    pred = _tc_mlp(emb, feat_vals, w, fmb, layer_params, w_out, b_out, blk=512)
    return pred.reshape(-1)


# speedup vs baseline: 1.6829x; 1.0315x over previous
import functools

import jax
import jax.numpy as jnp
from jax import lax
from jax.experimental import pallas as pl
from jax.experimental.pallas import tpu as pltpu
from jax.experimental.pallas import tpu_sc as plsc

_NUM_CORES = 2
_NUM_SUBCORES = 16
_NW = _NUM_CORES * _NUM_SUBCORES


_CB = 7


@functools.partial(jax.jit, static_argnames=("nb", "d"))
def _sc_detile(x4, *, nb, d):
    d2 = d // 8
    per_w = nb // _NW
    n_chunks = per_w // _CB
    chunk_elems = _CB * 128 * d
    mesh = plsc.VectorSubcoreMesh(core_axis_name="c", subcore_axis_name="s")

    @functools.partial(
        pl.kernel,
        mesh=mesh,
        compiler_params=pltpu.CompilerParams(
            use_tc_tiling_on_sc=False, needs_layout_passes=False),
        out_type=jax.ShapeDtypeStruct((nb * 128 * d,), jnp.float32),
        scratch_types=[
            pltpu.VMEM((chunk_elems,), jnp.float32),
            pltpu.VMEM((chunk_elems,), jnp.float32),
            pltpu.VMEM((chunk_elems,), jnp.float32),
            pltpu.VMEM((chunk_elems,), jnp.float32),
            pltpu.SemaphoreType.DMA,
            pltpu.SemaphoreType.DMA,
            pltpu.SemaphoreType.DMA,
        ],
    )
    def k(x4_hbm, out_hbm, in0, in1, out0, out1, sem_in, sem_o0, sem_o1):
        wid = lax.axis_index("s") * _NUM_CORES + lax.axis_index("c")
        cb_base = wid * per_w
        lane16 = lax.broadcasted_iota(jnp.int32, (16,), 0) * d

        def start_in(j, buf):
            cb0 = cb_base + j * _CB
            for a in range(d2):
                pltpu.async_copy(x4_hbm.at[pl.ds((a * nb + cb0) * 1024, _CB * 1024)],
                                 buf.at[pl.ds(a * _CB * 1024, _CB * 1024)], sem_in)

        def wait_in(buf):
            for a in range(d2):
                pltpu.make_async_copy(x4_hbm.at[pl.ds(0, _CB * 1024)],
                                      buf.at[pl.ds(a * _CB * 1024, _CB * 1024)],
                                      sem_in).wait()

        def regroup(buf, obuf):
            @functools.partial(plsc.parallel_loop, 0, d2 * _CB * 8, unroll=2)
            def _(t):
                a = t // (_CB * 8)
                ci = (t // 8) % _CB
                s = t % 8
                off = t * 128
                base = ci * (128 * d) + (8 * a + s)
                vecs = [buf[pl.ds(off + lg * 16, 16)] for lg in range(8)]
                for lg in range(8):
                    plsc.store_scatter(
                        obuf, [lane16 + (base + lg * 16 * d)], vecs[lg])

        def start_out(j, obuf, sem):
            cb0 = cb_base + j * _CB
            pltpu.async_copy(obuf, out_hbm.at[pl.ds(cb0 * 128 * d, chunk_elems)], sem)

        def wait_out(obuf, sem):
            pltpu.make_async_copy(obuf, out_hbm.at[pl.ds(0, chunk_elems)], sem).wait()

        start_in(0, in0)

        def pair_body(k2, carry):
            j0 = 2 * k2
            j1 = j0 + 1

            @pl.when(j1 < n_chunks)
            def _():
                start_in(j1, in1)
            wait_in(in0)

            @pl.when(j0 >= 2)
            def _():
                wait_out(out0, sem_o0)
            regroup(in0, out0)
            start_out(j0, out0, sem_o0)

            @pl.when(j1 < n_chunks)
            def _():
                @pl.when(j1 + 1 < n_chunks)
                def _():
                    start_in(j1 + 1, in0)
                wait_in(in1)

                @pl.when(j1 >= 2)
                def _():
                    wait_out(out1, sem_o1)
                regroup(in1, out1)
                start_out(j1, out1, sem_o1)
            return carry

        lax.fori_loop(0, (n_chunks + 1) // 2, pair_body, 0)
        wait_out(out0, sem_o0)
        if n_chunks >= 2:
            wait_out(out1, sem_o1)

    return k(x4)


@functools.partial(jax.jit, static_argnames=("n", "d", "chunk"))
def _sc_gather(ids, fm_v_lin, fm_w, *, n, d, chunk):
    fm_v = fm_v_lin.reshape(fm_v_lin.shape[0] // d, d)
    n_chunks = (n // _NW) // chunk
    per_w = n // _NW
    mesh = plsc.VectorSubcoreMesh(core_axis_name="c", subcore_axis_name="s")

    @functools.partial(
        pl.kernel,
        mesh=mesh,
        compiler_params=pltpu.CompilerParams(use_tc_tiling_on_sc=False),
        out_type=[
            jax.ShapeDtypeStruct((n, d), jnp.float32),
            jax.ShapeDtypeStruct((n,), jnp.float32),
        ],
        scratch_types=[
            pltpu.VMEM((chunk,), jnp.int32),
            pltpu.VMEM((chunk,), jnp.int32),
            pltpu.VMEM((chunk, d), jnp.float32),
            pltpu.VMEM((chunk, d), jnp.float32),
            pltpu.VMEM((chunk,), jnp.float32),
            pltpu.VMEM((chunk,), jnp.float32),
            pltpu.SemaphoreType.DMA,
            pltpu.SemaphoreType.DMA,
            pltpu.SemaphoreType.DMA,
            pltpu.SemaphoreType.DMA,
        ],
    )
    def k(ids_hbm, fmv_hbm, fmw_hbm, emb_out, w_out,
          idx0, idx1, r0, r1, w0, w1, sem_i, sem_g, sem_o0, sem_o1):
        wid = lax.axis_index("s") * _NUM_CORES + lax.axis_index("c")
        base = wid * per_w
        idxb, rb, wb = [idx0, idx1], [r0, r1], [w0, w1]
        sem_ob = [sem_o0, sem_o1]

        def off(j):
            return base + j * chunk

        def wait_idx(s):
            pltpu.make_async_copy(ids_hbm.at[pl.ds(0, chunk)], idxb[s], sem_i).wait()

        def wait_gather(s):
            pltpu.make_async_copy(fmv_hbm.at[idxb[s]], rb[s], sem_g).wait()
            pltpu.make_async_copy(fmw_hbm.at[idxb[s]], wb[s], sem_g).wait()

        def start_out(j, s):
            pltpu.async_copy(rb[s], emb_out.at[pl.ds(off(j), chunk)], sem_ob[s])
            pltpu.async_copy(wb[s], w_out.at[pl.ds(off(j), chunk)], sem_ob[s])

        def wait_out(s):
            pltpu.make_async_copy(rb[s], emb_out.at[pl.ds(0, chunk)], sem_ob[s]).wait()
            pltpu.make_async_copy(wb[s], w_out.at[pl.ds(0, chunk)], sem_ob[s]).wait()

        pltpu.async_copy(ids_hbm.at[pl.ds(off(0), chunk)], idx0, sem_i)
        for j in range(n_chunks):
            s = j % 2
            wait_idx(s)
            if j >= 2:
                wait_out(s)
            pltpu.async_copy(fmv_hbm.at[idxb[s]], rb[s], sem_g)
            pltpu.async_copy(fmw_hbm.at[idxb[s]], wb[s], sem_g)
            if j + 1 < n_chunks:
                pltpu.async_copy(ids_hbm.at[pl.ds(off(j + 1), chunk)], idxb[1 - s], sem_i)
            if j > 0:
                wait_gather(1 - s)
                start_out(j - 1, 1 - s)
        last = n_chunks - 1
        wait_gather(last % 2)
        start_out(last, last % 2)
        if n_chunks >= 2:
            wait_out((last - 1) % 2)
        wait_out(last % 2)

    return k(ids, fm_v, fm_w)


def _tc_body(n_layers, f, d, emb_ref, vals_ref, w_ref, fmb_ref, *param_refs):
    out_ref = param_refs[-1]
    param_refs = param_refs[:-1]
    fd = f * d
    emb = emb_ref[...]
    vals = vals_ref[...]

    rr = lax.broadcasted_iota(jnp.int32, (f, fd), 0)
    cc = lax.broadcasted_iota(jnp.int32, (f, fd), 1)
    s_mat = (cc // d == rr).astype(jnp.float32)
    vrep = jnp.dot(vals, s_mat, preferred_element_type=jnp.float32)
    x = emb * vrep

    c2 = lax.broadcasted_iota(jnp.int32, (fd, d), 0)
    d2 = lax.broadcasted_iota(jnp.int32, (fd, d), 1)
    t_mat = (c2 % d == d2).astype(jnp.float32)
    sum_vec = jnp.dot(x, t_mat, preferred_element_type=jnp.float32)
    y_v = 0.5 * (jnp.sum(sum_vec * sum_vec, axis=1, keepdims=True)
                 - jnp.sum(x * x, axis=1, keepdims=True))

    y_w = jnp.sum(w_ref[...] * vals, axis=1, keepdims=True)

    h = x
    for i in range(n_layers):
        w_l, b_l, inv_l, sh_l = param_refs[4 * i:4 * i + 4]
        h = jnp.dot(h.astype(jnp.bfloat16), w_l[...].astype(jnp.bfloat16),
                    preferred_element_type=jnp.float32) + b_l[...]
        h = jnp.maximum(h, 0.0)
        h = h * inv_l[...] + sh_l[...]
    w_out_ref, b_out_ref = param_refs[4 * n_layers:4 * n_layers + 2]
    y_d = jnp.dot(h.astype(jnp.bfloat16), w_out_ref[...].astype(jnp.bfloat16),
                  preferred_element_type=jnp.float32) + b_out_ref[0, 0]

    y = fmb_ref[0, 0] + y_w + y_v + y_d
    out_ref[...] = jax.nn.sigmoid(y)


def _tc_mlp(emb, vals, w, fmb, layer_params, w_out, b_out, *, blk):
    b_total, fd = emb.shape
    f = vals.shape[1]
    d = fd // f
    n_layers = len(layer_params)
    grid = (b_total // blk,)

    def row_spec(width):
        return pl.BlockSpec((blk, width), lambda i: (i, 0))

    def full_spec(shape):
        return pl.BlockSpec(shape, lambda i: (0,) * len(shape))

    in_specs = [
        row_spec(fd),
        row_spec(f),
        row_spec(f),
        full_spec((1, 1)),
    ]
    args = [emb, vals, w, fmb]
    for (w_l, b_l, inv_l, sh_l) in layer_params:
        in_specs += [full_spec(w_l.shape), full_spec(b_l.shape),
                     full_spec(inv_l.shape), full_spec(sh_l.shape)]
        args += [w_l, b_l, inv_l, sh_l]
    in_specs += [full_spec(w_out.shape), full_spec(b_out.shape)]
    args += [w_out, b_out]

    return pl.pallas_call(
        functools.partial(_tc_body, n_layers, f, d),
        grid=grid,
        in_specs=in_specs,
        out_specs=pl.BlockSpec((blk, 1), lambda i: (i, 0)),
        out_shape=jax.ShapeDtypeStruct((b_total, 1), jnp.float32),
    )(*args)


def kernel(feat_ids, feat_vals, FM_B, FM_W, FM_V, params):
    b, f = feat_ids.shape
    v, d = FM_V.shape
    n = b * f

    ids = feat_ids.reshape(-1).astype(jnp.int32)
    nb = ((v + 128 * _NW * _CB - 1) // (128 * _NW * _CB)) * _NW * _CB
    v_pad = nb * 128
    fvp = jnp.pad(FM_V, ((0, v_pad - v), (0, 0)))
    x1 = fvp.T.reshape(d // 8, 8, nb, 128).transpose(0, 2, 1, 3).reshape(-1)
    table_lin = _sc_detile(x1, nb=nb, d=d)
    emb_flat, w_flat = _sc_gather(ids, table_lin, FM_W, n=n, d=d, chunk=1664)
    emb = emb_flat.reshape(b, f * d)
    w = w_flat.reshape(b, f)

    n_layers = sum(1 for k in params if k.startswith("W") and k != "W_out")
    layer_params = []
    for i in range(n_layers):
        inv = params[f"gamma{i}"] / jnp.sqrt(params[f"var{i}"] + 1e-3)
        sh = params[f"beta{i}"] - params[f"mean{i}"] * inv
        layer_params.append((
            params[f"W{i}"],
            params[f"b{i}"].reshape(1, -1),
            inv.reshape(1, -1),
            sh.reshape(1, -1),
        ))
    w_out = params["W_out"]
    b_out = params["b_out"].reshape(1, 1)
    fmb = FM_B.reshape(1, 1)

    pred = _tc_mlp(emb, feat_vals, w, fmb, layer_params, w_out, b_out, blk=512)
    return pred.reshape(-1)


# speedup vs baseline: 1.7221x; 1.0233x over previous
import functools

import jax
import jax.numpy as jnp
from jax import lax
from jax.experimental import pallas as pl
from jax.experimental.pallas import tpu as pltpu
from jax.experimental.pallas import tpu_sc as plsc

_NUM_CORES = 2
_NUM_SUBCORES = 16
_NW = _NUM_CORES * _NUM_SUBCORES


_CB = 7


@functools.partial(jax.jit, static_argnames=("nb", "d"))
def _sc_detile(x4, *, nb, d):
    d2 = d // 8
    per_w = nb // _NW
    n_chunks = per_w // _CB
    chunk_elems = _CB * 128 * d
    mesh = plsc.VectorSubcoreMesh(core_axis_name="c", subcore_axis_name="s")

    @functools.partial(
        pl.kernel,
        mesh=mesh,
        compiler_params=pltpu.CompilerParams(
            use_tc_tiling_on_sc=False, needs_layout_passes=False),
        out_type=jax.ShapeDtypeStruct((nb * 128 * d,), jnp.float32),
        scratch_types=[
            pltpu.VMEM((chunk_elems,), jnp.float32),
            pltpu.VMEM((chunk_elems,), jnp.float32),
            pltpu.VMEM((chunk_elems,), jnp.float32),
            pltpu.VMEM((chunk_elems,), jnp.float32),
            pltpu.SemaphoreType.DMA,
            pltpu.SemaphoreType.DMA,
            pltpu.SemaphoreType.DMA,
        ],
    )
    def k(x4_hbm, out_hbm, in0, in1, out0, out1, sem_in, sem_o0, sem_o1):
        wid = lax.axis_index("s") * _NUM_CORES + lax.axis_index("c")
        cb_base = wid * per_w
        lane16 = lax.broadcasted_iota(jnp.int32, (16,), 0) * d

        def start_in(j, buf):
            cb0 = cb_base + j * _CB
            for a in range(d2):
                pltpu.async_copy(x4_hbm.at[pl.ds((a * nb + cb0) * 1024, _CB * 1024)],
                                 buf.at[pl.ds(a * _CB * 1024, _CB * 1024)], sem_in)

        def wait_in(buf):
            for a in range(d2):
                pltpu.make_async_copy(x4_hbm.at[pl.ds(0, _CB * 1024)],
                                      buf.at[pl.ds(a * _CB * 1024, _CB * 1024)],
                                      sem_in).wait()

        def regroup(buf, obuf):
            @functools.partial(plsc.parallel_loop, 0, d2 * _CB * 8, unroll=2)
            def _(t):
                a = t // (_CB * 8)
                ci = (t // 8) % _CB
                s = t % 8
                off = t * 128
                base = ci * (128 * d) + (8 * a + s)
                vecs = [buf[pl.ds(off + lg * 16, 16)] for lg in range(8)]
                for lg in range(8):
                    plsc.store_scatter(
                        obuf, [lane16 + (base + lg * 16 * d)], vecs[lg])

        def start_out(j, obuf, sem):
            cb0 = cb_base + j * _CB
            pltpu.async_copy(obuf, out_hbm.at[pl.ds(cb0 * 128 * d, chunk_elems)], sem)

        def wait_out(obuf, sem):
            pltpu.make_async_copy(obuf, out_hbm.at[pl.ds(0, chunk_elems)], sem).wait()

        start_in(0, in0)

        def pair_body(k2, carry):
            j0 = 2 * k2
            j1 = j0 + 1

            @pl.when(j1 < n_chunks)
            def _():
                start_in(j1, in1)
            wait_in(in0)

            @pl.when(j0 >= 2)
            def _():
                wait_out(out0, sem_o0)
            regroup(in0, out0)
            start_out(j0, out0, sem_o0)

            @pl.when(j1 < n_chunks)
            def _():
                @pl.when(j1 + 1 < n_chunks)
                def _():
                    start_in(j1 + 1, in0)
                wait_in(in1)

                @pl.when(j1 >= 2)
                def _():
                    wait_out(out1, sem_o1)
                regroup(in1, out1)
                start_out(j1, out1, sem_o1)
            return carry

        lax.fori_loop(0, (n_chunks + 1) // 2, pair_body, 0)
        wait_out(out0, sem_o0)
        if n_chunks >= 2:
            wait_out(out1, sem_o1)

    return k(x4)


@functools.partial(jax.jit, static_argnames=("n", "d", "chunk"))
def _sc_gather(ids, fm_v_lin, fm_w, *, n, d, chunk):
    fm_v = fm_v_lin.reshape(fm_v_lin.shape[0] // d, d)
    n_chunks = (n // _NW) // chunk
    per_w = n // _NW
    mesh = plsc.VectorSubcoreMesh(core_axis_name="c", subcore_axis_name="s")

    @functools.partial(
        pl.kernel,
        mesh=mesh,
        compiler_params=pltpu.CompilerParams(use_tc_tiling_on_sc=False),
        out_type=[
            jax.ShapeDtypeStruct((n, d), jnp.float32),
            jax.ShapeDtypeStruct((n,), jnp.float32),
        ],
        scratch_types=[
            pltpu.VMEM((chunk,), jnp.int32),
            pltpu.VMEM((chunk,), jnp.int32),
            pltpu.VMEM((chunk, d), jnp.float32),
            pltpu.VMEM((chunk, d), jnp.float32),
            pltpu.VMEM((chunk,), jnp.float32),
            pltpu.VMEM((chunk,), jnp.float32),
            pltpu.SemaphoreType.DMA,
            pltpu.SemaphoreType.DMA,
            pltpu.SemaphoreType.DMA,
            pltpu.SemaphoreType.DMA,
        ],
    )
    def k(ids_hbm, fmv_hbm, fmw_hbm, emb_out, w_out,
          idx0, idx1, r0, r1, w0, w1, sem_i, sem_g, sem_o0, sem_o1):
        wid = lax.axis_index("s") * _NUM_CORES + lax.axis_index("c")
        base = wid * per_w
        idxb, rb, wb = [idx0, idx1], [r0, r1], [w0, w1]
        sem_ob = [sem_o0, sem_o1]

        def off(j):
            return base + j * chunk

        def wait_idx(s):
            pltpu.make_async_copy(ids_hbm.at[pl.ds(0, chunk)], idxb[s], sem_i).wait()

        def wait_gather(s):
            pltpu.make_async_copy(fmv_hbm.at[idxb[s]], rb[s], sem_g).wait()
            pltpu.make_async_copy(fmw_hbm.at[idxb[s]], wb[s], sem_g).wait()

        def start_out(j, s):
            pltpu.async_copy(rb[s], emb_out.at[pl.ds(off(j), chunk)], sem_ob[s])
            pltpu.async_copy(wb[s], w_out.at[pl.ds(off(j), chunk)], sem_ob[s])

        def wait_out(s):
            pltpu.make_async_copy(rb[s], emb_out.at[pl.ds(0, chunk)], sem_ob[s]).wait()
            pltpu.make_async_copy(wb[s], w_out.at[pl.ds(0, chunk)], sem_ob[s]).wait()

        pltpu.async_copy(ids_hbm.at[pl.ds(off(0), chunk)], idx0, sem_i)
        for j in range(n_chunks):
            s = j % 2
            wait_idx(s)
            if j >= 2:
                wait_out(s)
            pltpu.async_copy(fmv_hbm.at[idxb[s]], rb[s], sem_g)
            pltpu.async_copy(fmw_hbm.at[idxb[s]], wb[s], sem_g)
            if j + 1 < n_chunks:
                pltpu.async_copy(ids_hbm.at[pl.ds(off(j + 1), chunk)], idxb[1 - s], sem_i)
            if j > 0:
                wait_gather(1 - s)
                start_out(j - 1, 1 - s)
        last = n_chunks - 1
        wait_gather(last % 2)
        start_out(last, last % 2)
        if n_chunks >= 2:
            wait_out((last - 1) % 2)
        wait_out(last % 2)

    return k(ids, fm_v, fm_w)


def _tc_body(n_layers, f, d, emb_ref, vals_ref, w_ref, fmb_ref, *param_refs):
    out_ref = param_refs[-1]
    param_refs = param_refs[:-1]
    fd = f * d
    emb = emb_ref[...]
    vals = vals_ref[...]

    rr = lax.broadcasted_iota(jnp.int32, (f, fd), 0)
    cc = lax.broadcasted_iota(jnp.int32, (f, fd), 1)
    s_mat = (cc // d == rr).astype(jnp.bfloat16)
    vrep = jnp.dot(vals.astype(jnp.bfloat16), s_mat,
                   preferred_element_type=jnp.float32)
    x = emb * vrep

    c2 = lax.broadcasted_iota(jnp.int32, (fd, d), 0)
    d2 = lax.broadcasted_iota(jnp.int32, (fd, d), 1)
    t_mat = (c2 % d == d2).astype(jnp.float32)
    sum_vec = jnp.dot(x, t_mat, preferred_element_type=jnp.float32)
    y_v = 0.5 * (jnp.sum(sum_vec * sum_vec, axis=1, keepdims=True)
                 - jnp.sum(x * x, axis=1, keepdims=True))

    y_w = jnp.sum(w_ref[...] * vals, axis=1, keepdims=True)

    h = x
    for i in range(n_layers):
        w_l, b_l, inv_l, sh_l = param_refs[4 * i:4 * i + 4]
        h = jnp.dot(h.astype(jnp.bfloat16), w_l[...].astype(jnp.bfloat16),
                    preferred_element_type=jnp.float32) + b_l[...]
        h = jnp.maximum(h, 0.0)
        h = h * inv_l[...] + sh_l[...]
    w_out_ref, b_out_ref = param_refs[4 * n_layers:4 * n_layers + 2]
    y_d = jnp.dot(h.astype(jnp.bfloat16), w_out_ref[...].astype(jnp.bfloat16),
                  preferred_element_type=jnp.float32) + b_out_ref[0, 0]

    y = fmb_ref[0, 0] + y_w + y_v + y_d
    out_ref[...] = jax.nn.sigmoid(y)


def _tc_mlp(emb, vals, w, fmb, layer_params, w_out, b_out, *, blk):
    b_total, fd = emb.shape
    f = vals.shape[1]
    d = fd // f
    n_layers = len(layer_params)
    grid = (b_total // blk,)

    def row_spec(width):
        return pl.BlockSpec((blk, width), lambda i: (i, 0))

    def full_spec(shape):
        return pl.BlockSpec(shape, lambda i: (0,) * len(shape))

    in_specs = [
        row_spec(fd),
        row_spec(f),
        row_spec(f),
        full_spec((1, 1)),
    ]
    args = [emb, vals, w, fmb]
    for (w_l, b_l, inv_l, sh_l) in layer_params:
        in_specs += [full_spec(w_l.shape), full_spec(b_l.shape),
                     full_spec(inv_l.shape), full_spec(sh_l.shape)]
        args += [w_l, b_l, inv_l, sh_l]
    in_specs += [full_spec(w_out.shape), full_spec(b_out.shape)]
    args += [w_out, b_out]

    return pl.pallas_call(
        functools.partial(_tc_body, n_layers, f, d),
        grid=grid,
        in_specs=in_specs,
        out_specs=pl.BlockSpec((blk, 1), lambda i: (i, 0)),
        out_shape=jax.ShapeDtypeStruct((b_total, 1), jnp.float32),
    )(*args)


def kernel(feat_ids, feat_vals, FM_B, FM_W, FM_V, params):
    b, f = feat_ids.shape
    v, d = FM_V.shape
    n = b * f

    ids = feat_ids.reshape(-1).astype(jnp.int32)
    nb = ((v + 128 * _NW * _CB - 1) // (128 * _NW * _CB)) * _NW * _CB
    v_pad = nb * 128
    fvp = jnp.pad(FM_V, ((0, v_pad - v), (0, 0)))
    x1 = fvp.T.reshape(d // 8, 8, nb, 128).transpose(0, 2, 1, 3).reshape(-1)
    table_lin = _sc_detile(x1, nb=nb, d=d)
    emb_flat, w_flat = _sc_gather(ids, table_lin, FM_W, n=n, d=d, chunk=1664)
    emb = emb_flat.reshape(b, f * d)
    w = w_flat.reshape(b, f)

    n_layers = sum(1 for k in params if k.startswith("W") and k != "W_out")
    layer_params = []
    for i in range(n_layers):
        inv = params[f"gamma{i}"] / jnp.sqrt(params[f"var{i}"] + 1e-3)
        sh = params[f"beta{i}"] - params[f"mean{i}"] * inv
        layer_params.append((
            params[f"W{i}"],
            params[f"b{i}"].reshape(1, -1),
            inv.reshape(1, -1),
            sh.reshape(1, -1),
        ))
    w_out = params["W_out"]
    b_out = params["b_out"].reshape(1, 1)
    fmb = FM_B.reshape(1, 1)

    pred = _tc_mlp(emb, feat_vals, w, fmb, layer_params, w_out, b_out, blk=1024)
    return pred.reshape(-1)
